# Initial kernel scaffold; baseline (speedup 1.0000x reference)
import jax
import jax.numpy as jnp
from jax.experimental import pallas as pl


def kernel(X, emb_E, emb_E_var, emb_TE, alpha_E, beta_E, omega_E, emb_R, emb_R_var, emb_TR, alpha_R, beta_R, omega_R):
    raise NotImplementedError("write your pallas kernel here")


# speedup vs baseline: 1.5891x; 1.5891x over previous
import jax
import jax.numpy as jnp
from jax import lax
from jax.experimental import pallas as pl
from jax.experimental.pallas import tpu as pltpu
from jax.experimental.pallas import tpu_sc as plsc

N_ENT = 100000
N_REL = 1000
D = 64
B = 16384

NC = 2
NS = 16
NW = NC * NS
NPW = B // NW
C = 64
NCHUNK = NPW // C

S1 = 6.2831826
S3 = -41.341423
S5 = 81.59618
S7 = -76.5801
S9 = 41.205395
S11 = -12.271261
MAGIC = 12582912.0


def _sin2pi(y):
    n = (y + MAGIC) - MAGIC
    t = y - n
    u = t * t
    p = S11
    for c in (S9, S7, S5, S3, S1):
        p = p * u + c
    return p * t


def _body(h_hbm, t_hbm, r_hbm, d_hbm,
          eE_hbm, eEv_hbm, eTE_hbm, aE_hbm, bE_hbm, oE_hbm,
          eR_hbm, eRv_hbm, eTR_hbm, aR_hbm, bR_hbm, oR_hbm,
          out_hbm,
          ht_v, r_v, d_v,
          eE_v, eEv_v, eTE_v, bE_v, oE_v, aE_v,
          eR_v, eRv_v, eTR_v, bR_v, oR_v, aR_v,
          out_v, sem):
    wid = lax.axis_index("s") * NC + lax.axis_index("c")
    base = wid * NPW
    iota16 = jnp.arange(16, dtype=jnp.int32)

    for chunk in range(NCHUNK):
        cb = base + chunk * C
        pltpu.sync_copy(h_hbm.at[pl.ds(cb, C)], ht_v.at[pl.ds(0, C)])
        pltpu.sync_copy(t_hbm.at[pl.ds(cb, C)], ht_v.at[pl.ds(C, C)])
        pltpu.sync_copy(r_hbm.at[pl.ds(cb, C)], r_v)
        pltpu.sync_copy(d_hbm.at[pl.ds(cb, C)], d_v.at[pl.ds(0, C)])

        cps = []
        for tbl, dst in ((eE_hbm, eE_v), (eEv_hbm, eEv_v), (eTE_hbm, eTE_v),
                         (bE_hbm, bE_v), (oE_hbm, oE_v)):
            cps.append(pltpu.async_copy(tbl.at[ht_v], dst, sem))
        cps.append(pltpu.async_copy(aE_hbm.at[ht_v], aE_v.at[pl.ds(0, 2 * C)],
                                    sem))
        for tbl, dst in ((eR_hbm, eR_v), (eRv_hbm, eRv_v), (eTR_hbm, eTR_v),
                         (bR_hbm, bR_v), (oR_hbm, oR_v)):
            cps.append(pltpu.async_copy(tbl.at[r_v], dst, sem))
        cps.append(pltpu.async_copy(aR_hbm.at[r_v], aR_v.at[pl.ds(0, C)], sem))
        for cp in cps:
            cp.wait()

        def triple_body(i, outvec, chunk=chunk):
            ti = i + C
            d_s = d_v[pl.ds(i, 16)][0]
            sh = d_s * aE_v[pl.ds(i, 16)][0]
            st = d_s * aE_v[pl.ds(ti, 16)][0]
            sr = d_s * aR_v[pl.ds(i, 16)][0]
            acc = jnp.zeros((16,), jnp.float32)
            for k in range(D // 16):
                sl = pl.ds(16 * k, 16)
                he = eE_v[i, sl]
                te = eE_v[ti, sl]
                hv = eEv_v[i, sl]
                tv = eEv_v[ti, sl]
                hte = eTE_v[i, sl]
                tte = eTE_v[ti, sl]
                hb = bE_v[i, sl]
                tb = bE_v[ti, sl]
                ho = oE_v[i, sl]
                to = oE_v[ti, sl]
                re = eR_v[i, sl]
                rv = eRv_v[i, sl]
                rte = eTR_v[i, sl]
                rb = bR_v[i, sl]
                ro = oR_v[i, sl]

                hm = he + sh * hte + hb * _sin2pi(ho * d_s)
                tm = te + st * tte + tb * _sin2pi(to * d_s)
                rm = re + sr * rte + rb * _sin2pi(ro * d_s)
                s = hv + tv
                q = rm - hm + tm
                q = q * q
                acc = acc + (s + q) / rv + (rv + q) / s
            tot = acc[0]
            for l in range(1, 16):
                tot = tot + acc[l]
            tot = (tot - 2.0 * D) * 0.25
            lane = jnp.bitwise_and(i, 15)
            outvec = jnp.where(iota16 == lane, tot, outvec)

            @pl.when(lane == 15)
            def _store(outvec=outvec, i=i):
                out_v[pl.ds(chunk * C + i - 15, 16)] = outvec

            return outvec

        lax.fori_loop(0, C, triple_body, jnp.zeros((16,), jnp.float32))

    pltpu.sync_copy(out_v, out_hbm.at[pl.ds(base, NPW)])


@jax.jit
def _run(h_i, t_i, r_i, d_f,
         emb_E, emb_E_var, emb_TE, alpha_E, beta_E, omega_E,
         emb_R, emb_R_var, emb_TR, alpha_R, beta_R, omega_R):
    mesh = plsc.VectorSubcoreMesh(core_axis_name="c", subcore_axis_name="s",
                                  num_cores=NC, num_subcores=NS)
    k = pl.kernel(
        _body,
        out_type=jax.ShapeDtypeStruct((B,), jnp.float32),
        mesh=mesh,
        scratch_types=[
            pltpu.VMEM((2 * C,), jnp.int32),
            pltpu.VMEM((C,), jnp.int32),
            pltpu.VMEM((C + 16,), jnp.float32),
            pltpu.VMEM((2 * C, D), jnp.float32),
            pltpu.VMEM((2 * C, D), jnp.float32),
            pltpu.VMEM((2 * C, D), jnp.float32),
            pltpu.VMEM((2 * C, D), jnp.float32),
            pltpu.VMEM((2 * C, D), jnp.float32),
            pltpu.VMEM((2 * C + 16,), jnp.float32),
            pltpu.VMEM((C, D), jnp.float32),
            pltpu.VMEM((C, D), jnp.float32),
            pltpu.VMEM((C, D), jnp.float32),
            pltpu.VMEM((C, D), jnp.float32),
            pltpu.VMEM((C, D), jnp.float32),
            pltpu.VMEM((C + 16,), jnp.float32),
            pltpu.VMEM((NPW,), jnp.float32),
            pltpu.SemaphoreType.DMA,
        ],
        compiler_params=pltpu.CompilerParams(use_tc_tiling_on_sc=False),
        interpret=False,
    )
    return k(h_i, t_i, r_i, d_f,
             emb_E, emb_E_var, emb_TE, alpha_E, beta_E, omega_E,
             emb_R, emb_R_var, emb_TR, alpha_R, beta_R, omega_R)


def kernel(X, emb_E, emb_E_var, emb_TE, alpha_E, beta_E, omega_E,
           emb_R, emb_R_var, emb_TR, alpha_R, beta_R, omega_R):
    h_i = X[:, 0]
    t_i = X[:, 1]
    r_i = X[:, 2]
    d_f = X[:, 3].astype(jnp.float32)
    return _run(h_i, t_i, r_i, d_f,
                emb_E, emb_E_var, emb_TE, alpha_E.reshape(-1), beta_E, omega_E,
                emb_R, emb_R_var, emb_TR, alpha_R.reshape(-1), beta_R, omega_R)


# speedup vs baseline: 1.7210x; 1.0830x over previous
import jax
import jax.numpy as jnp
from jax import lax
from jax.experimental import pallas as pl
from jax.experimental.pallas import tpu as pltpu
from jax.experimental.pallas import tpu_sc as plsc

N_ENT = 100000
N_REL = 1000
D = 64
B = 16384

NC = 2
NS = 16
NW = NC * NS
NPW = B // NW
C = 64
NCHUNK = NPW // C

S1 = 6.2831826
S3 = -41.341423
S5 = 81.59618
S7 = -76.5801
S9 = 41.205395
S11 = -12.271261
MAGIC = 12582912.0


def _sin2pi(y):
    n = (y + MAGIC) - MAGIC
    t = y - n
    u = t * t
    p = S11
    for c in (S9, S7, S5, S3, S1):
        p = p * u + c
    return p * t


_SLOT_TYPES = [
    pltpu.VMEM((2 * C,), jnp.int32),
    pltpu.VMEM((C,), jnp.int32),
    pltpu.VMEM((C + 16,), jnp.float32),
    pltpu.VMEM((2 * C, D), jnp.float32),
    pltpu.VMEM((2 * C, D), jnp.float32),
    pltpu.VMEM((2 * C, D), jnp.float32),
    pltpu.VMEM((2 * C, D), jnp.float32),
    pltpu.VMEM((2 * C, D), jnp.float32),
    pltpu.VMEM((2 * C + 16,), jnp.float32),
    pltpu.VMEM((C, D), jnp.float32),
    pltpu.VMEM((C, D), jnp.float32),
    pltpu.VMEM((C, D), jnp.float32),
    pltpu.VMEM((C, D), jnp.float32),
    pltpu.VMEM((C, D), jnp.float32),
    pltpu.VMEM((C + 16,), jnp.float32),
    pltpu.SemaphoreType.DMA,
]
_NS_SLOT = len(_SLOT_TYPES)


def _body(h_hbm, t_hbm, r_hbm, d_hbm,
          eE_hbm, eEv_hbm, eTE_hbm, aE_hbm, bE_hbm, oE_hbm,
          eR_hbm, eRv_hbm, eTR_hbm, aR_hbm, bR_hbm, oR_hbm,
          out_hbm, *scratch):
    slots = (scratch[:_NS_SLOT], scratch[_NS_SLOT:2 * _NS_SLOT])
    out_v = scratch[2 * _NS_SLOT]
    wid = lax.axis_index("s") * NC + lax.axis_index("c")
    base = wid * NPW
    iota16 = jnp.arange(16, dtype=jnp.int32)

    def start(chunk):
        (ht_v, r_v, d_v, eE_v, eEv_v, eTE_v, bE_v, oE_v, aE_v,
         eR_v, eRv_v, eTR_v, bR_v, oR_v, aR_v, sem) = slots[chunk % 2]
        cb = base + chunk * C
        pltpu.sync_copy(h_hbm.at[pl.ds(cb, C)], ht_v.at[pl.ds(0, C)])
        pltpu.sync_copy(t_hbm.at[pl.ds(cb, C)], ht_v.at[pl.ds(C, C)])
        pltpu.sync_copy(r_hbm.at[pl.ds(cb, C)], r_v)
        pltpu.sync_copy(d_hbm.at[pl.ds(cb, C)], d_v.at[pl.ds(0, C)])

        cps = []
        for tbl, dst in ((eE_hbm, eE_v), (eEv_hbm, eEv_v), (eTE_hbm, eTE_v),
                         (bE_hbm, bE_v), (oE_hbm, oE_v)):
            cps.append(pltpu.async_copy(tbl.at[ht_v], dst, sem))
        cps.append(pltpu.async_copy(aE_hbm.at[ht_v], aE_v.at[pl.ds(0, 2 * C)],
                                    sem))
        for tbl, dst in ((eR_hbm, eR_v), (eRv_hbm, eRv_v), (eTR_hbm, eTR_v),
                         (bR_hbm, bR_v), (oR_hbm, oR_v)):
            cps.append(pltpu.async_copy(tbl.at[r_v], dst, sem))
        cps.append(pltpu.async_copy(aR_hbm.at[r_v], aR_v.at[pl.ds(0, C)], sem))
        return cps

    def compute(chunk):
        (ht_v, r_v, d_v, eE_v, eEv_v, eTE_v, bE_v, oE_v, aE_v,
         eR_v, eRv_v, eTR_v, bR_v, oR_v, aR_v, sem) = slots[chunk % 2]

        def triple_body(i, outvec):
            ti = i + C
            d_s = d_v[pl.ds(i, 16)][0]
            sh = d_s * aE_v[pl.ds(i, 16)][0]
            st = d_s * aE_v[pl.ds(ti, 16)][0]
            sr = d_s * aR_v[pl.ds(i, 16)][0]
            acc = jnp.zeros((16,), jnp.float32)
            for k in range(D // 16):
                sl = pl.ds(16 * k, 16)
                he = eE_v[i, sl]
                te = eE_v[ti, sl]
                hv = eEv_v[i, sl]
                tv = eEv_v[ti, sl]
                hte = eTE_v[i, sl]
                tte = eTE_v[ti, sl]
                hb = bE_v[i, sl]
                tb = bE_v[ti, sl]
                ho = oE_v[i, sl]
                to = oE_v[ti, sl]
                re = eR_v[i, sl]
                rv = eRv_v[i, sl]
                rte = eTR_v[i, sl]
                rb = bR_v[i, sl]
                ro = oR_v[i, sl]

                hm = he + sh * hte + hb * _sin2pi(ho * d_s)
                tm = te + st * tte + tb * _sin2pi(to * d_s)
                rm = re + sr * rte + rb * _sin2pi(ro * d_s)
                s = hv + tv
                q = rm - hm + tm
                q = q * q
                acc = acc + (s + q) / rv + (rv + q) / s
            tot = acc[0]
            for l in range(1, 16):
                tot = tot + acc[l]
            tot = (tot - 2.0 * D) * 0.25
            lane = jnp.bitwise_and(i, 15)
            outvec = jnp.where(iota16 == lane, tot, outvec)

            @pl.when(lane == 15)
            def _store(outvec=outvec, i=i):
                out_v[pl.ds(chunk * C + i - 15, 16)] = outvec

            return outvec

        lax.fori_loop(0, C, triple_body, jnp.zeros((16,), jnp.float32))

    cps = start(0)
    for c in range(NCHUNK):
        for cp in cps:
            cp.wait()
        if c + 1 < NCHUNK:
            cps = start(c + 1)
        compute(c)

    pltpu.sync_copy(out_v, out_hbm.at[pl.ds(base, NPW)])


@jax.jit
def _run(h_i, t_i, r_i, d_f,
         emb_E, emb_E_var, emb_TE, alpha_E, beta_E, omega_E,
         emb_R, emb_R_var, emb_TR, alpha_R, beta_R, omega_R):
    mesh = plsc.VectorSubcoreMesh(core_axis_name="c", subcore_axis_name="s",
                                  num_cores=NC, num_subcores=NS)
    k = pl.kernel(
        _body,
        out_type=jax.ShapeDtypeStruct((B,), jnp.float32),
        mesh=mesh,
        scratch_types=_SLOT_TYPES + _SLOT_TYPES + [
            pltpu.VMEM((NPW,), jnp.float32),
        ],
        compiler_params=pltpu.CompilerParams(use_tc_tiling_on_sc=False),
        interpret=False,
    )
    return k(h_i, t_i, r_i, d_f,
             emb_E, emb_E_var, emb_TE, alpha_E, beta_E, omega_E,
             emb_R, emb_R_var, emb_TR, alpha_R, beta_R, omega_R)


def kernel(X, emb_E, emb_E_var, emb_TE, alpha_E, beta_E, omega_E,
           emb_R, emb_R_var, emb_TR, alpha_R, beta_R, omega_R):
    h_i = X[:, 0]
    t_i = X[:, 1]
    r_i = X[:, 2]
    d_f = X[:, 3].astype(jnp.float32)
    return _run(h_i, t_i, r_i, d_f,
                emb_E, emb_E_var, emb_TE, alpha_E.reshape(-1), beta_E, omega_E,
                emb_R, emb_R_var, emb_TR, alpha_R.reshape(-1), beta_R, omega_R)


# speedup vs baseline: 1.9263x; 1.1193x over previous
import jax
import jax.numpy as jnp
from jax import lax
from jax.experimental import pallas as pl
from jax.experimental.pallas import tpu as pltpu
from jax.experimental.pallas import tpu_sc as plsc

N_ENT = 100000
N_REL = 1000
D = 64
D2 = 2 * D
B = 16384

NC = 2
NS = 16
NW = NC * NS
NPW = B // NW
C = 64
NCHUNK = NPW // C

S1 = 6.2831826
S3 = -41.341423
S5 = 81.59618
S7 = -76.5801
S9 = 41.205395
S11 = -12.271261
MAGIC = 12582912.0


def _sin2pi(y):
    n = (y + MAGIC) - MAGIC
    t = y - n
    u = t * t
    p = S11
    for c in (S9, S7, S5, S3, S1):
        p = p * u + c
    return p * t


_SLOT_TYPES = [
    pltpu.VMEM((2 * C,), jnp.int32),
    pltpu.VMEM((C,), jnp.int32),
    pltpu.VMEM((C + 16,), jnp.float32),
    pltpu.VMEM((2 * C, D2), jnp.float32),
    pltpu.VMEM((2 * C, D2), jnp.float32),
    pltpu.VMEM((2 * C, D), jnp.float32),
    pltpu.VMEM((2 * C + 16,), jnp.float32),
    pltpu.VMEM((C, D2), jnp.float32),
    pltpu.VMEM((C, D2), jnp.float32),
    pltpu.VMEM((C, D), jnp.float32),
    pltpu.VMEM((C + 16,), jnp.float32),
    pltpu.SemaphoreType.DMA,
]
_NS_SLOT = len(_SLOT_TYPES)


def _body(h_hbm, t_hbm, r_hbm, d_hbm,
          p1_hbm, p2_hbm, eEv_hbm, aE_hbm,
          q1_hbm, q2_hbm, eRv_hbm, aR_hbm,
          out_hbm, *scratch):
    slots = (scratch[:_NS_SLOT], scratch[_NS_SLOT:2 * _NS_SLOT])
    out_v = scratch[2 * _NS_SLOT]
    wid = lax.axis_index("s") * NC + lax.axis_index("c")
    base = wid * NPW
    iota16 = jnp.arange(16, dtype=jnp.int32)

    def start(chunk):
        (ht_v, r_v, d_v, p1_v, p2_v, eEv_v, aE_v,
         q1_v, q2_v, eRv_v, aR_v, sem) = slots[chunk % 2]
        cb = base + chunk * C
        pltpu.sync_copy(h_hbm.at[pl.ds(cb, C)], ht_v.at[pl.ds(0, C)])
        pltpu.sync_copy(t_hbm.at[pl.ds(cb, C)], ht_v.at[pl.ds(C, C)])
        pltpu.sync_copy(r_hbm.at[pl.ds(cb, C)], r_v)
        pltpu.sync_copy(d_hbm.at[pl.ds(cb, C)], d_v.at[pl.ds(0, C)])

        cps = [
            pltpu.async_copy(p1_hbm.at[ht_v], p1_v, sem),
            pltpu.async_copy(p2_hbm.at[ht_v], p2_v, sem),
            pltpu.async_copy(eEv_hbm.at[ht_v], eEv_v, sem),
            pltpu.async_copy(aE_hbm.at[ht_v], aE_v.at[pl.ds(0, 2 * C)], sem),
            pltpu.async_copy(q1_hbm.at[r_v], q1_v, sem),
            pltpu.async_copy(q2_hbm.at[r_v], q2_v, sem),
            pltpu.async_copy(eRv_hbm.at[r_v], eRv_v, sem),
            pltpu.async_copy(aR_hbm.at[r_v], aR_v.at[pl.ds(0, C)], sem),
        ]
        return cps

    def compute(chunk):
        (ht_v, r_v, d_v, p1_v, p2_v, eEv_v, aE_v,
         q1_v, q2_v, eRv_v, aR_v, sem) = slots[chunk % 2]

        def triple_body(i, outvec):
            ti = i + C
            d_s = d_v[pl.ds(i, 16)][0]
            sh = d_s * aE_v[pl.ds(i, 16)][0]
            st = d_s * aE_v[pl.ds(ti, 16)][0]
            sr = d_s * aR_v[pl.ds(i, 16)][0]
            acc = jnp.zeros((16,), jnp.float32)
            for k in range(D // 16):
                sl = pl.ds(16 * k, 16)
                s2 = pl.ds(D + 16 * k, 16)
                he = p1_v[i, sl]
                hte = p1_v[i, s2]
                te = p1_v[ti, sl]
                tte = p1_v[ti, s2]
                hb = p2_v[i, sl]
                ho = p2_v[i, s2]
                tb = p2_v[ti, sl]
                to = p2_v[ti, s2]
                hv = eEv_v[i, sl]
                tv = eEv_v[ti, sl]
                re = q1_v[i, sl]
                rte = q1_v[i, s2]
                rb = q2_v[i, sl]
                ro = q2_v[i, s2]
                rv = eRv_v[i, sl]

                hm = he + sh * hte + hb * _sin2pi(ho * d_s)
                tm = te + st * tte + tb * _sin2pi(to * d_s)
                rm = re + sr * rte + rb * _sin2pi(ro * d_s)
                s = hv + tv
                q = rm - hm + tm
                q = q * q
                acc = acc + (s + q) / rv + (rv + q) / s
            tot = acc[0]
            for l in range(1, 16):
                tot = tot + acc[l]
            tot = (tot - 2.0 * D) * 0.25
            lane = jnp.bitwise_and(i, 15)
            outvec = jnp.where(iota16 == lane, tot, outvec)

            @pl.when(lane == 15)
            def _store(outvec=outvec, i=i):
                out_v[pl.ds(chunk * C + i - 15, 16)] = outvec

            return outvec

        lax.fori_loop(0, C, triple_body, jnp.zeros((16,), jnp.float32))

    cps = start(0)
    for c in range(NCHUNK):
        for cp in cps:
            cp.wait()
        if c + 1 < NCHUNK:
            cps = start(c + 1)
        compute(c)

    pltpu.sync_copy(out_v, out_hbm.at[pl.ds(base, NPW)])


@jax.jit
def _run(h_i, t_i, r_i, d_f,
         emb_E, emb_E_var, emb_TE, alpha_E, beta_E, omega_E,
         emb_R, emb_R_var, emb_TR, alpha_R, beta_R, omega_R):
    p1 = jnp.concatenate([emb_E, emb_TE], axis=1)
    p2 = jnp.concatenate([beta_E, omega_E], axis=1)
    q1 = jnp.concatenate([emb_R, emb_TR], axis=1)
    q2 = jnp.concatenate([beta_R, omega_R], axis=1)
    mesh = plsc.VectorSubcoreMesh(core_axis_name="c", subcore_axis_name="s",
                                  num_cores=NC, num_subcores=NS)
    k = pl.kernel(
        _body,
        out_type=jax.ShapeDtypeStruct((B,), jnp.float32),
        mesh=mesh,
        scratch_types=_SLOT_TYPES + _SLOT_TYPES + [
            pltpu.VMEM((NPW,), jnp.float32),
        ],
        compiler_params=pltpu.CompilerParams(use_tc_tiling_on_sc=False),
        interpret=False,
    )
    return k(h_i, t_i, r_i, d_f,
             p1, p2, emb_E_var, alpha_E,
             q1, q2, emb_R_var, alpha_R)


def kernel(X, emb_E, emb_E_var, emb_TE, alpha_E, beta_E, omega_E,
           emb_R, emb_R_var, emb_TR, alpha_R, beta_R, omega_R):
    h_i = X[:, 0]
    t_i = X[:, 1]
    r_i = X[:, 2]
    d_f = X[:, 3].astype(jnp.float32)
    return _run(h_i, t_i, r_i, d_f,
                emb_E, emb_E_var, emb_TE, alpha_E.reshape(-1), beta_E, omega_E,
                emb_R, emb_R_var, emb_TR, alpha_R.reshape(-1), beta_R, omega_R)


# speedup vs baseline: 1.9477x; 1.0111x over previous
import jax
import jax.numpy as jnp
from jax import lax
from jax.experimental import pallas as pl
from jax.experimental.pallas import tpu as pltpu
from jax.experimental.pallas import tpu_sc as plsc

N_ENT = 100000
N_REL = 1000
D = 64
D2 = 2 * D
B = 16384

NC = 2
NS = 16
NW = NC * NS
NPW = B // NW
C = 64
NCHUNK = NPW // C

S1 = 6.2831826
S3 = -41.341423
S5 = 81.59618
S7 = -76.5801
S9 = 41.205395
S11 = -12.271261
MAGIC = 12582912.0


def _sin2pi(y):
    n = (y + MAGIC) - MAGIC
    t = y - n
    u = t * t
    p = S11
    for c in (S9, S7, S5, S3, S1):
        p = p * u + c
    return p * t


_SLOT_TYPES = [
    pltpu.VMEM((2 * C,), jnp.int32),
    pltpu.VMEM((C,), jnp.int32),
    pltpu.VMEM((C + 16,), jnp.float32),
    pltpu.VMEM((2 * C, D2), jnp.float32),
    pltpu.VMEM((2 * C, D2), jnp.float32),
    pltpu.VMEM((2 * C, D), jnp.float32),
    pltpu.VMEM((2 * C + 16,), jnp.float32),
    pltpu.VMEM((C, D2), jnp.float32),
    pltpu.VMEM((C, D2), jnp.float32),
    pltpu.VMEM((C, D), jnp.float32),
    pltpu.VMEM((C + 16,), jnp.float32),
    pltpu.SemaphoreType.DMA,
]
_NS_SLOT = len(_SLOT_TYPES)


def _body(h_hbm, t_hbm, r_hbm, d_hbm,
          p1_hbm, p2_hbm, eEv_hbm, aE_hbm,
          q1_hbm, q2_hbm, eRv_hbm, aR_hbm,
          out_hbm, *scratch):
    slots = (scratch[:_NS_SLOT], scratch[_NS_SLOT:2 * _NS_SLOT])
    out_v = scratch[2 * _NS_SLOT]
    wid = lax.axis_index("s") * NC + lax.axis_index("c")
    base = wid * NPW
    iota16 = jnp.arange(16, dtype=jnp.int32)

    def start(chunk):
        (ht_v, r_v, d_v, p1_v, p2_v, eEv_v, aE_v,
         q1_v, q2_v, eRv_v, aR_v, sem) = slots[chunk % 2]
        cb = base + chunk * C
        pltpu.sync_copy(h_hbm.at[pl.ds(cb, C)], ht_v.at[pl.ds(0, C)])
        pltpu.sync_copy(t_hbm.at[pl.ds(cb, C)], ht_v.at[pl.ds(C, C)])
        pltpu.sync_copy(r_hbm.at[pl.ds(cb, C)], r_v)
        pltpu.sync_copy(d_hbm.at[pl.ds(cb, C)], d_v.at[pl.ds(0, C)])

        cps = [
            pltpu.async_copy(p1_hbm.at[ht_v], p1_v, sem),
            pltpu.async_copy(p2_hbm.at[ht_v], p2_v, sem),
            pltpu.async_copy(eEv_hbm.at[ht_v], eEv_v, sem),
            pltpu.async_copy(aE_hbm.at[ht_v], aE_v.at[pl.ds(0, 2 * C)], sem),
            pltpu.async_copy(q1_hbm.at[r_v], q1_v, sem),
            pltpu.async_copy(q2_hbm.at[r_v], q2_v, sem),
            pltpu.async_copy(eRv_hbm.at[r_v], eRv_v, sem),
            pltpu.async_copy(aR_hbm.at[r_v], aR_v.at[pl.ds(0, C)], sem),
        ]
        return cps

    def compute(chunk):
        (ht_v, r_v, d_v, p1_v, p2_v, eEv_v, aE_v,
         q1_v, q2_v, eRv_v, aR_v, sem) = slots[chunk % 2]

        def triple_body(i, outvec):
            ti = i + C
            d_s = d_v[pl.ds(i, 16)][0]
            sh = d_s * aE_v[pl.ds(i, 16)][0]
            st = d_s * aE_v[pl.ds(ti, 16)][0]
            sr = d_s * aR_v[pl.ds(i, 16)][0]
            acc = jnp.zeros((16,), jnp.float32)
            for k in range(D // 16):
                sl = pl.ds(16 * k, 16)
                s2 = pl.ds(D + 16 * k, 16)
                he = p1_v[i, sl]
                hte = p1_v[i, s2]
                te = p1_v[ti, sl]
                tte = p1_v[ti, s2]
                hb = p2_v[i, sl]
                ho = p2_v[i, s2]
                tb = p2_v[ti, sl]
                to = p2_v[ti, s2]
                hv = eEv_v[i, sl]
                tv = eEv_v[ti, sl]
                re = q1_v[i, sl]
                rte = q1_v[i, s2]
                rb = q2_v[i, sl]
                ro = q2_v[i, s2]
                rv = eRv_v[i, sl]

                hm = he + sh * hte + hb * _sin2pi(ho * d_s)
                tm = te + st * tte + tb * _sin2pi(to * d_s)
                rm = re + sr * rte + rb * _sin2pi(ro * d_s)
                s = hv + tv
                q = rm - hm + tm
                q = q * q
                acc = acc + (s * s + rv * rv + q * (s + rv)) / (rv * s)
            tot = acc[0]
            for l in range(1, 16):
                tot = tot + acc[l]
            tot = (tot - 2.0 * D) * 0.25
            lane = jnp.bitwise_and(i, 15)
            outvec = jnp.where(iota16 == lane, tot, outvec)

            @pl.when(lane == 15)
            def _store(outvec=outvec, i=i):
                out_v[pl.ds(chunk * C + i - 15, 16)] = outvec

            return outvec

        lax.fori_loop(0, C, triple_body, jnp.zeros((16,), jnp.float32))

    cps = start(0)
    for c in range(NCHUNK):
        for cp in cps:
            cp.wait()
        if c + 1 < NCHUNK:
            cps = start(c + 1)
        compute(c)

    pltpu.sync_copy(out_v, out_hbm.at[pl.ds(base, NPW)])


@jax.jit
def _run(h_i, t_i, r_i, d_f,
         emb_E, emb_E_var, emb_TE, alpha_E, beta_E, omega_E,
         emb_R, emb_R_var, emb_TR, alpha_R, beta_R, omega_R):
    p1 = jnp.concatenate([emb_E, emb_TE], axis=1)
    p2 = jnp.concatenate([beta_E, omega_E], axis=1)
    q1 = jnp.concatenate([emb_R, emb_TR], axis=1)
    q2 = jnp.concatenate([beta_R, omega_R], axis=1)
    mesh = plsc.VectorSubcoreMesh(core_axis_name="c", subcore_axis_name="s",
                                  num_cores=NC, num_subcores=NS)
    k = pl.kernel(
        _body,
        out_type=jax.ShapeDtypeStruct((B,), jnp.float32),
        mesh=mesh,
        scratch_types=_SLOT_TYPES + _SLOT_TYPES + [
            pltpu.VMEM((NPW,), jnp.float32),
        ],
        compiler_params=pltpu.CompilerParams(use_tc_tiling_on_sc=False),
        interpret=False,
    )
    return k(h_i, t_i, r_i, d_f,
             p1, p2, emb_E_var, alpha_E,
             q1, q2, emb_R_var, alpha_R)


def kernel(X, emb_E, emb_E_var, emb_TE, alpha_E, beta_E, omega_E,
           emb_R, emb_R_var, emb_TR, alpha_R, beta_R, omega_R):
    h_i = X[:, 0]
    t_i = X[:, 1]
    r_i = X[:, 2]
    d_f = X[:, 3].astype(jnp.float32)
    return _run(h_i, t_i, r_i, d_f,
                emb_E, emb_E_var, emb_TE, alpha_E.reshape(-1), beta_E, omega_E,
                emb_R, emb_R_var, emb_TR, alpha_R.reshape(-1), beta_R, omega_R)


# speedup vs baseline: 1.9963x; 1.0250x over previous
import jax
import jax.numpy as jnp
from jax import lax
from jax.experimental import pallas as pl
from jax.experimental.pallas import tpu as pltpu
from jax.experimental.pallas import tpu_sc as plsc

N_ENT = 100000
N_REL = 1000
D = 64
D2 = 2 * D
B = 16384

NC = 2
NS = 16
NW = NC * NS
NPW = B // NW
C = 64
NCHUNK = NPW // C

S1 = 6.2831826
S3 = -41.341423
S5 = 81.59618
S7 = -76.5801
S9 = 41.205395
S11 = -12.271261
MAGIC = 12582912.0


def _sin2pi(y):
    n = (y + MAGIC) - MAGIC
    t = y - n
    u = t * t
    p = S11
    for c in (S9, S7, S5, S3, S1):
        p = p * u + c
    return p * t


_SLOT_TYPES = [
    pltpu.VMEM((2 * C,), jnp.int32),
    pltpu.VMEM((C,), jnp.int32),
    pltpu.VMEM((C + 16,), jnp.float32),
    pltpu.VMEM((2 * C, D2), jnp.float32),
    pltpu.VMEM((2 * C, D2), jnp.float32),
    pltpu.VMEM((2 * C, D), jnp.float32),
    pltpu.VMEM((2 * C + 16,), jnp.float32),
    pltpu.VMEM((C, D2), jnp.float32),
    pltpu.VMEM((C, D2), jnp.float32),
    pltpu.VMEM((C, D), jnp.float32),
    pltpu.VMEM((C + 16,), jnp.float32),
    pltpu.SemaphoreType.DMA,
]
_NS_SLOT = len(_SLOT_TYPES)


def _body(h_hbm, t_hbm, r_hbm, d_hbm,
          p1_hbm, p2_hbm, eEv_hbm, aE_hbm,
          q1_hbm, q2_hbm, eRv_hbm, aR_hbm,
          out_hbm, *scratch):
    slots = (scratch[:_NS_SLOT], scratch[_NS_SLOT:2 * _NS_SLOT])
    out_v = scratch[2 * _NS_SLOT]
    wid = lax.axis_index("s") * NC + lax.axis_index("c")
    base = wid * NPW
    iota16 = jnp.arange(16, dtype=jnp.int32)

    def start(chunk):
        (ht_v, r_v, d_v, p1_v, p2_v, eEv_v, aE_v,
         q1_v, q2_v, eRv_v, aR_v, sem) = slots[chunk % 2]
        cb = base + chunk * C
        pltpu.sync_copy(h_hbm.at[pl.ds(cb, C)], ht_v.at[pl.ds(0, C)])
        pltpu.sync_copy(t_hbm.at[pl.ds(cb, C)], ht_v.at[pl.ds(C, C)])
        pltpu.sync_copy(r_hbm.at[pl.ds(cb, C)], r_v)
        pltpu.sync_copy(d_hbm.at[pl.ds(cb, C)], d_v.at[pl.ds(0, C)])

        cps = [
            pltpu.async_copy(p1_hbm.at[ht_v], p1_v, sem),
            pltpu.async_copy(p2_hbm.at[ht_v], p2_v, sem),
            pltpu.async_copy(eEv_hbm.at[ht_v], eEv_v, sem),
            pltpu.async_copy(aE_hbm.at[ht_v], aE_v.at[pl.ds(0, 2 * C)], sem),
            pltpu.async_copy(q1_hbm.at[r_v], q1_v, sem),
            pltpu.async_copy(q2_hbm.at[r_v], q2_v, sem),
            pltpu.async_copy(eRv_hbm.at[r_v], eRv_v, sem),
            pltpu.async_copy(aR_hbm.at[r_v], aR_v.at[pl.ds(0, C)], sem),
        ]
        return cps

    def compute(chunk):
        (ht_v, r_v, d_v, p1_v, p2_v, eEv_v, aE_v,
         q1_v, q2_v, eRv_v, aR_v, sem) = slots[chunk % 2]

        def triple_body(i, outvec):
            ti = i + C
            d_s = d_v[pl.ds(i, 16)][0]
            sh = d_s * aE_v[pl.ds(i, 16)][0]
            st = d_s * aE_v[pl.ds(ti, 16)][0]
            sr = d_s * aR_v[pl.ds(i, 16)][0]
            acc = jnp.zeros((16,), jnp.float32)
            for k in range(D // 16):
                sl = pl.ds(16 * k, 16)
                s2 = pl.ds(D + 16 * k, 16)
                he = p1_v[i, sl]
                hte = p1_v[i, s2]
                te = p1_v[ti, sl]
                tte = p1_v[ti, s2]
                hb = p2_v[i, sl]
                ho = p2_v[i, s2]
                tb = p2_v[ti, sl]
                to = p2_v[ti, s2]
                hv = eEv_v[i, sl]
                tv = eEv_v[ti, sl]
                re = q1_v[i, sl]
                rte = q1_v[i, s2]
                rb = q2_v[i, sl]
                ro = q2_v[i, s2]
                rv = eRv_v[i, sl]

                hm = he + sh * hte + hb * _sin2pi(ho * d_s)
                tm = te + st * tte + tb * _sin2pi(to * d_s)
                rm = re + sr * rte + rb * _sin2pi(ro * d_s)
                s = hv + tv
                q = rm - hm + tm
                q = q * q
                acc = acc + (s * s + rv * rv + q * (s + rv)) / (rv * s)
            tot = acc[0]
            for l in range(1, 16):
                tot = tot + acc[l]
            tot = (tot - 2.0 * D) * 0.25
            lane = jnp.bitwise_and(i, 15)
            outvec = jnp.where(iota16 == lane, tot, outvec)

            @pl.when(lane == 15)
            def _store(outvec=outvec, i=i):
                out_v[pl.ds(chunk * C + i - 15, 16)] = outvec

            return outvec

        lax.fori_loop(0, C, triple_body, jnp.zeros((16,), jnp.float32),
                      unroll=2)

    cps = start(0)
    for c in range(NCHUNK):
        for cp in cps:
            cp.wait()
        if c + 1 < NCHUNK:
            cps = start(c + 1)
        compute(c)

    pltpu.sync_copy(out_v, out_hbm.at[pl.ds(base, NPW)])


@jax.jit
def _run(h_i, t_i, r_i, d_f,
         emb_E, emb_E_var, emb_TE, alpha_E, beta_E, omega_E,
         emb_R, emb_R_var, emb_TR, alpha_R, beta_R, omega_R):
    p1 = jnp.concatenate([emb_E, emb_TE], axis=1)
    p2 = jnp.concatenate([beta_E, omega_E], axis=1)
    q1 = jnp.concatenate([emb_R, emb_TR], axis=1)
    q2 = jnp.concatenate([beta_R, omega_R], axis=1)
    mesh = plsc.VectorSubcoreMesh(core_axis_name="c", subcore_axis_name="s",
                                  num_cores=NC, num_subcores=NS)
    k = pl.kernel(
        _body,
        out_type=jax.ShapeDtypeStruct((B,), jnp.float32),
        mesh=mesh,
        scratch_types=_SLOT_TYPES + _SLOT_TYPES + [
            pltpu.VMEM((NPW,), jnp.float32),
        ],
        compiler_params=pltpu.CompilerParams(use_tc_tiling_on_sc=False),
        interpret=False,
    )
    return k(h_i, t_i, r_i, d_f,
             p1, p2, emb_E_var, alpha_E,
             q1, q2, emb_R_var, alpha_R)


def kernel(X, emb_E, emb_E_var, emb_TE, alpha_E, beta_E, omega_E,
           emb_R, emb_R_var, emb_TR, alpha_R, beta_R, omega_R):
    h_i = X[:, 0]
    t_i = X[:, 1]
    r_i = X[:, 2]
    d_f = X[:, 3].astype(jnp.float32)
    return _run(h_i, t_i, r_i, d_f,
                emb_E, emb_E_var, emb_TE, alpha_E.reshape(-1), beta_E, omega_E,
                emb_R, emb_R_var, emb_TR, alpha_R.reshape(-1), beta_R, omega_R)


# speedup vs baseline: 2.0116x; 1.0076x over previous
import jax
import jax.numpy as jnp
from jax import lax
from jax.experimental import pallas as pl
from jax.experimental.pallas import tpu as pltpu
from jax.experimental.pallas import tpu_sc as plsc

N_ENT = 100000
N_REL = 1000
D = 64
D2 = 2 * D
B = 16384

NC = 2
NS = 16
NW = NC * NS
NPW = B // NW
C = 64
NCHUNK = NPW // C

S1 = 6.2831826
S3 = -41.341423
S5 = 81.59618
S7 = -76.5801
S9 = 41.205395
S11 = -12.271261
MAGIC = 12582912.0


def _sin2pi(y):
    n = (y + MAGIC) - MAGIC
    t = y - n
    u = t * t
    p = S11
    for c in (S9, S7, S5, S3, S1):
        p = p * u + c
    return p * t


_SLOT_TYPES = [
    pltpu.VMEM((2 * C,), jnp.int32),
    pltpu.VMEM((C,), jnp.int32),
    pltpu.VMEM((C + 16,), jnp.float32),
    pltpu.VMEM((2 * C, D2), jnp.float32),
    pltpu.VMEM((2 * C, D2), jnp.float32),
    pltpu.VMEM((2 * C, D), jnp.float32),
    pltpu.VMEM((2 * C + 16,), jnp.float32),
    pltpu.VMEM((C, D2), jnp.float32),
    pltpu.VMEM((C, D2), jnp.float32),
    pltpu.VMEM((C, D), jnp.float32),
    pltpu.VMEM((C + 16,), jnp.float32),
    pltpu.SemaphoreType.DMA,
]
_NS_SLOT = len(_SLOT_TYPES)


def _body(h_hbm, t_hbm, r_hbm, d_hbm,
          p1_hbm, p2_hbm, eEv_hbm, aE_hbm,
          q1_hbm, q2_hbm, eRv_hbm, aR_hbm,
          out_hbm, *scratch):
    slots = (scratch[:_NS_SLOT], scratch[_NS_SLOT:2 * _NS_SLOT])
    out_v = scratch[2 * _NS_SLOT]
    wid = lax.axis_index("s") * NC + lax.axis_index("c")
    base = wid * NPW
    iota16 = jnp.arange(16, dtype=jnp.int32)

    def start(chunk):
        (ht_v, r_v, d_v, p1_v, p2_v, eEv_v, aE_v,
         q1_v, q2_v, eRv_v, aR_v, sem) = slots[chunk % 2]
        cb = base + chunk * C
        pltpu.sync_copy(h_hbm.at[pl.ds(cb, C)], ht_v.at[pl.ds(0, C)])
        pltpu.sync_copy(t_hbm.at[pl.ds(cb, C)], ht_v.at[pl.ds(C, C)])
        pltpu.sync_copy(r_hbm.at[pl.ds(cb, C)], r_v)
        pltpu.sync_copy(d_hbm.at[pl.ds(cb, C)], d_v.at[pl.ds(0, C)])

        cps = [
            pltpu.async_copy(p1_hbm.at[ht_v], p1_v, sem),
            pltpu.async_copy(p2_hbm.at[ht_v], p2_v, sem),
            pltpu.async_copy(eEv_hbm.at[ht_v], eEv_v, sem),
            pltpu.async_copy(aE_hbm.at[ht_v], aE_v.at[pl.ds(0, 2 * C)], sem),
            pltpu.async_copy(q1_hbm.at[r_v], q1_v, sem),
            pltpu.async_copy(q2_hbm.at[r_v], q2_v, sem),
            pltpu.async_copy(eRv_hbm.at[r_v], eRv_v, sem),
            pltpu.async_copy(aR_hbm.at[r_v], aR_v.at[pl.ds(0, C)], sem),
        ]
        return cps

    def compute(chunk):
        (ht_v, r_v, d_v, p1_v, p2_v, eEv_v, aE_v,
         q1_v, q2_v, eRv_v, aR_v, sem) = slots[chunk % 2]

        def triple_body(i, outvec):
            ti = i + C
            d_s = d_v[pl.ds(i, 16)][0]
            sh = d_s * aE_v[pl.ds(i, 16)][0]
            st = d_s * aE_v[pl.ds(ti, 16)][0]
            sr = d_s * aR_v[pl.ds(i, 16)][0]
            acc = jnp.zeros((16,), jnp.float32)
            for k in range(D // 16):
                sl = pl.ds(16 * k, 16)
                s2 = pl.ds(D + 16 * k, 16)
                he = p1_v[i, sl]
                hte = p1_v[i, s2]
                te = p1_v[ti, sl]
                tte = p1_v[ti, s2]
                hb = p2_v[i, sl]
                ho = p2_v[i, s2]
                tb = p2_v[ti, sl]
                to = p2_v[ti, s2]
                hv = eEv_v[i, sl]
                tv = eEv_v[ti, sl]
                re = q1_v[i, sl]
                rte = q1_v[i, s2]
                rb = q2_v[i, sl]
                ro = q2_v[i, s2]
                rv = eRv_v[i, sl]

                hm = he + sh * hte + hb * _sin2pi(ho * d_s)
                tm = te + st * tte + tb * _sin2pi(to * d_s)
                rm = re + sr * rte + rb * _sin2pi(ro * d_s)
                s = hv + tv
                q = rm - hm + tm
                q = q * q
                acc = acc + (s * s + rv * rv + q * (s + rv)) / (rv * s)
            tot = acc[0]
            for l in range(1, 16):
                tot = tot + acc[l]
            tot = (tot - 2.0 * D) * 0.25
            lane = jnp.bitwise_and(i, 15)
            outvec = jnp.where(iota16 == lane, tot, outvec)

            @pl.when(lane == 15)
            def _store(outvec=outvec, i=i):
                out_v[pl.ds(chunk * C + i - 15, 16)] = outvec

            return outvec

        lax.fori_loop(0, C, triple_body, jnp.zeros((16,), jnp.float32),
                      unroll=4)

    cps = start(0)
    for c in range(NCHUNK):
        for cp in cps:
            cp.wait()
        if c + 1 < NCHUNK:
            cps = start(c + 1)
        compute(c)

    pltpu.sync_copy(out_v, out_hbm.at[pl.ds(base, NPW)])


@jax.jit
def _run(h_i, t_i, r_i, d_f,
         emb_E, emb_E_var, emb_TE, alpha_E, beta_E, omega_E,
         emb_R, emb_R_var, emb_TR, alpha_R, beta_R, omega_R):
    p1 = jnp.concatenate([emb_E, emb_TE], axis=1)
    p2 = jnp.concatenate([beta_E, omega_E], axis=1)
    q1 = jnp.concatenate([emb_R, emb_TR], axis=1)
    q2 = jnp.concatenate([beta_R, omega_R], axis=1)
    mesh = plsc.VectorSubcoreMesh(core_axis_name="c", subcore_axis_name="s",
                                  num_cores=NC, num_subcores=NS)
    k = pl.kernel(
        _body,
        out_type=jax.ShapeDtypeStruct((B,), jnp.float32),
        mesh=mesh,
        scratch_types=_SLOT_TYPES + _SLOT_TYPES + [
            pltpu.VMEM((NPW,), jnp.float32),
        ],
        compiler_params=pltpu.CompilerParams(use_tc_tiling_on_sc=False),
        interpret=False,
    )
    return k(h_i, t_i, r_i, d_f,
             p1, p2, emb_E_var, alpha_E,
             q1, q2, emb_R_var, alpha_R)


def kernel(X, emb_E, emb_E_var, emb_TE, alpha_E, beta_E, omega_E,
           emb_R, emb_R_var, emb_TR, alpha_R, beta_R, omega_R):
    h_i = X[:, 0]
    t_i = X[:, 1]
    r_i = X[:, 2]
    d_f = X[:, 3].astype(jnp.float32)
    return _run(h_i, t_i, r_i, d_f,
                emb_E, emb_E_var, emb_TE, alpha_E.reshape(-1), beta_E, omega_E,
                emb_R, emb_R_var, emb_TR, alpha_R.reshape(-1), beta_R, omega_R)
